# Initial kernel scaffold; baseline (speedup 1.0000x reference)
import functools

import jax
import jax.numpy as jnp
from jax import lax
from jax.experimental import pallas as pl
from jax.experimental.pallas import tpu as pltpu
from jax.experimental.pallas import tpu_sc as plsc

_NC = 2
_NS = 16
_NW = _NC * _NS

_G = 128
_CHUNK = 512
_GPC = _CHUNK // _G


def _x_gather_sc(x_flat, w0):
    n = x_flat.shape[0]
    d = w0.shape[1]
    per_w = n // _NW
    k = per_w // _G
    n_chunks = per_w // _CHUNK

    x3 = x_flat.reshape(_NW, k, _G)

    mesh = plsc.VectorSubcoreMesh(core_axis_name="c", subcore_axis_name="s")

    @functools.partial(
        pl.kernel,
        out_type=jax.ShapeDtypeStruct((n, d), jnp.float32),
        mesh=mesh,
        scratch_types=[
            pltpu.VMEM((k, _G), jnp.int32),
            pltpu.VMEM((_CHUNK, d), jnp.float32),
            pltpu.SemaphoreType.DMA,
        ],
    )
    def gather_kernel(x_hbm, w0_hbm, out_hbm, idx_v, rows_v, sem):
        wid = lax.axis_index("s") * _NC + lax.axis_index("c")
        base = wid * per_w
        pltpu.sync_copy(x_hbm.at[wid], idx_v)

        def chunk_body(c, carry):
            copies = []
            for g in range(_GPC):
                copies.append(
                    pltpu.async_copy(
                        w0_hbm.at[idx_v.at[c * _GPC + g]],
                        rows_v.at[pl.ds(g * _G, _G)],
                        sem,
                    )
                )
            for cp in copies:
                cp.wait()
            pltpu.sync_copy(rows_v, out_hbm.at[pl.ds(base + c * _CHUNK, _CHUNK)])
            return carry

        lax.fori_loop(0, n_chunks, chunk_body, 0)

    return gather_kernel(x3, w0)


def _y_embed_tc(y_flat, w1):
    n = y_flat.shape[0]
    d = w1.shape[1]
    rows = 2048
    nb = n // rows
    y3 = y_flat.reshape(nb, 1, rows)
    w1p = jnp.pad(w1, ((0, 16 - w1.shape[0]), (0, 0)))

    def body(y_ref, w1_ref, o_ref):
        idx = y_ref[0, 0, :]
        oh = (idx[:, None] == lax.broadcasted_iota(jnp.int32, (rows, 16), 1))
        o_ref[...] = jnp.dot(
            oh.astype(jnp.float32), w1_ref[...],
            preferred_element_type=jnp.float32,
        )

    out = pl.pallas_call(
        body,
        grid=(nb,),
        in_specs=[
            pl.BlockSpec((1, 1, rows), lambda i: (i, 0, 0)),
            pl.BlockSpec((16, d), lambda i: (0, 0)),
        ],
        out_specs=pl.BlockSpec((rows, d), lambda i: (i, 0)),
        out_shape=jax.ShapeDtypeStruct((n, d), jnp.float32),
    )(y3, w1p)
    return out


def kernel(x, w0, y, w1):
    b, s = x.shape
    n = b * s
    x_emb = _x_gather_sc(x.reshape(n).astype(jnp.int32), w0)
    y_emb = _y_embed_tc(y.reshape(n).astype(jnp.int32), w1)
    return (x_emb.reshape(b, s, w0.shape[1]), y_emb.reshape(b, s, w1.shape[1]))


# speedup vs baseline: 1.8070x; 1.8070x over previous
import functools

import jax
import jax.numpy as jnp
from jax import lax
from jax.experimental import pallas as pl
from jax.experimental.pallas import tpu as pltpu
from jax.experimental.pallas import tpu_sc as plsc

_NC = 2
_NS = 16
_NW = _NC * _NS

_G = 128
_CHUNK = 512
_GPC = _CHUNK // _G


def _x_gather_sc(x_flat, w0):
    n = x_flat.shape[0]
    d = w0.shape[1]
    per_w = n // _NW
    k = per_w // _G
    n_chunks = per_w // _CHUNK

    x3 = x_flat.reshape(_NW, k, _G)

    mesh = plsc.VectorSubcoreMesh(core_axis_name="c", subcore_axis_name="s")

    @functools.partial(
        pl.kernel,
        out_type=jax.ShapeDtypeStruct((n, d), jnp.float32),
        mesh=mesh,
        compiler_params=pltpu.CompilerParams(use_tc_tiling_on_sc=False),
        scratch_types=[
            pltpu.VMEM((k, _G), jnp.int32),
            pltpu.VMEM((_CHUNK, d), jnp.float32),
            pltpu.SemaphoreType.DMA,
        ],
    )
    def gather_kernel(x_hbm, w0_hbm, out_hbm, idx_v, rows_v, sem):
        wid = lax.axis_index("s") * _NC + lax.axis_index("c")
        base = wid * per_w
        pltpu.sync_copy(x_hbm.at[wid], idx_v)

        def chunk_body(c, carry):
            copies = []
            for g in range(_GPC):
                copies.append(
                    pltpu.async_copy(
                        w0_hbm.at[idx_v.at[c * _GPC + g]],
                        rows_v.at[pl.ds(g * _G, _G)],
                        sem,
                    )
                )
            for cp in copies:
                cp.wait()
            pltpu.sync_copy(rows_v, out_hbm.at[pl.ds(base + c * _CHUNK, _CHUNK)])
            return carry

        lax.fori_loop(0, n_chunks, chunk_body, 0)

    return gather_kernel(x3, w0)


def _y_embed_tc(y_flat, w1):
    n = y_flat.shape[0]
    d = w1.shape[1]
    rows = 2048
    nb = n // rows
    y3 = y_flat.reshape(nb, 1, rows)
    w1p = jnp.pad(w1, ((0, 16 - w1.shape[0]), (0, 0)))

    def body(y_ref, w1_ref, o_ref):
        idx = y_ref[0, 0, :]
        oh = (idx[:, None] == lax.broadcasted_iota(jnp.int32, (rows, 16), 1))
        o_ref[...] = jnp.dot(
            oh.astype(jnp.float32), w1_ref[...],
            preferred_element_type=jnp.float32,
        )

    out = pl.pallas_call(
        body,
        grid=(nb,),
        in_specs=[
            pl.BlockSpec((1, 1, rows), lambda i: (i, 0, 0)),
            pl.BlockSpec((16, d), lambda i: (0, 0)),
        ],
        out_specs=pl.BlockSpec((rows, d), lambda i: (i, 0)),
        out_shape=jax.ShapeDtypeStruct((n, d), jnp.float32),
    )(y3, w1p)
    return out


def kernel(x, w0, y, w1):
    b, s = x.shape
    n = b * s
    x_emb = _x_gather_sc(x.reshape(n).astype(jnp.int32), w0)
    y_emb = _y_embed_tc(y.reshape(n).astype(jnp.int32), w1)
    return (x_emb.reshape(b, s, w0.shape[1]), y_emb.reshape(b, s, w1.shape[1]))
